# Initial kernel scaffold; baseline (speedup 1.0000x reference)
import jax
import jax.numpy as jnp
from jax.experimental import pallas as pl


def kernel(x, encoding):
    raise NotImplementedError("write your pallas kernel here")


# speedup vs baseline: 1.9450x; 1.9450x over previous
import jax
import jax.numpy as jnp
from jax.experimental import pallas as pl


def _add_kernel(x_ref, enc_ref, out_ref):
    out_ref[...] = x_ref[...] + enc_ref[...][None, :, :]


def kernel(x, encoding):
    N, S, D = x.shape
    BS = 512
    grid = (S // BS,)
    return pl.pallas_call(
        _add_kernel,
        grid=grid,
        in_specs=[
            pl.BlockSpec((N, BS, D), lambda i: (0, i, 0)),
            pl.BlockSpec((BS, D), lambda i: (i, 0)),
        ],
        out_specs=pl.BlockSpec((N, BS, D), lambda i: (0, i, 0)),
        out_shape=jax.ShapeDtypeStruct((N, S, D), x.dtype),
    )(x, encoding)
